# baseline (speedup 1.0000x reference)
import functools

import jax
import jax.numpy as jnp
from jax.experimental import pallas as pl
from jax.experimental.pallas import tpu as pltpu

_NUM_ENT = 10000
_BN_STEP = 128
_Q_ROWS = 32
_HT_ROW = 32
_IDX_ROWS = 40


def _gather_kernel(idx_hbm, table_ref, ht_out, rel_out, ent_out,
                   buf_a, buf_b, sem_a, sem_b, *, ppc, n_pairs):
    core = pl.program_id(0)
    j = pl.program_id(1)
    base = (core * ppc + j) * 2

    def start(step, buf, sem):
        pltpu.make_async_copy(idx_hbm.at[step], buf, sem).start()

    @pl.when(j == 0)
    def _():
        start(base, buf_a, sem_a)
        start(base + 1, buf_b, sem_b)

    def gather_half(buf, half):
        qrow0 = half * _BN_STEP * n_pairs
        hrow0 = half * _BN_STEP * 2
        for bn_l in range(_BN_STEP):
            for p in range(n_pairs):
                f = bn_l * 2 * n_pairs + 2 * p
                ridx = buf[f // 128, f % 128]
                eidx = buf[(f + 1) // 128, (f + 1) % 128]
                orow = qrow0 + bn_l * n_pairs + p
                rel_out[pl.ds(orow, 1)] = table_ref[pl.ds(ridx, 1)]
                ent_out[pl.ds(orow, 1)] = table_ref[pl.ds(eidx, 1)]
        for k in range(2 * _BN_STEP):
            hidx = buf[_HT_ROW + k // 128, k % 128]
            ht_out[pl.ds(hrow0 + k, 1)] = table_ref[pl.ds(hidx, 1)]

    pltpu.make_async_copy(idx_hbm.at[base], buf_a, sem_a).wait()
    gather_half(buf_a, 0)

    @pl.when(j + 1 < ppc)
    def _():
        start(base + 2, buf_a, sem_a)

    pltpu.make_async_copy(idx_hbm.at[base + 1], buf_b, sem_b).wait()
    gather_half(buf_b, 1)

    @pl.when(j + 1 < ppc)
    def _():
        start(base + 3, buf_b, sem_b)


def kernel(fused_table, ht_idx, qual_idx):
    v, es = fused_table.shape
    b, n, _ = ht_idx.shape
    q = qual_idx.shape[2]
    n_pairs = q // 2
    bn = b * n

    steps = bn // _BN_STEP
    assert bn % _BN_STEP == 0 and (_BN_STEP * q) == _Q_ROWS * 128
    assert (2 * _BN_STEP) % 128 == 0 and steps % 4 == 0
    ppc = steps // 4

    even = (jnp.arange(q) % 2) == 0
    q_off = qual_idx.astype(jnp.int32) + jnp.where(even, _NUM_ENT, 0).astype(jnp.int32)

    q_blk = q_off.reshape(steps, _Q_ROWS, 128)
    ht_rows = 2 * _BN_STEP // 128
    ht_blk = ht_idx.astype(jnp.int32).reshape(steps, ht_rows, 128)
    pad = jnp.zeros((steps, _IDX_ROWS - _Q_ROWS - ht_rows, 128), jnp.int32)
    idx_hbm = jnp.concatenate([q_blk, ht_blk, pad], axis=1)

    table3 = fused_table.reshape(v, 1, es)

    out_shape = [
        jax.ShapeDtypeStruct((bn * 2, 1, es), fused_table.dtype),
        jax.ShapeDtypeStruct((bn * n_pairs, 1, es), fused_table.dtype),
        jax.ShapeDtypeStruct((bn * n_pairs, 1, es), fused_table.dtype),
    ]
    ht_out, rel_out, ent_out = pl.pallas_call(
        functools.partial(_gather_kernel, ppc=ppc, n_pairs=n_pairs),
        grid=(2, ppc),
        in_specs=[
            pl.BlockSpec(memory_space=pl.ANY),
            pl.BlockSpec((v, 1, es), lambda c, j: (0, 0, 0)),
        ],
        out_specs=[
            pl.BlockSpec((4 * _BN_STEP, 1, es),
                         lambda c, j, ppc=ppc: (c * ppc + j, 0, 0)),
            pl.BlockSpec((2 * _BN_STEP * n_pairs, 1, es),
                         lambda c, j, ppc=ppc: (c * ppc + j, 0, 0)),
            pl.BlockSpec((2 * _BN_STEP * n_pairs, 1, es),
                         lambda c, j, ppc=ppc: (c * ppc + j, 0, 0)),
        ],
        out_shape=out_shape,
        scratch_shapes=[
            pltpu.SMEM((_IDX_ROWS, 128), jnp.int32),
            pltpu.SMEM((_IDX_ROWS, 128), jnp.int32),
            pltpu.SemaphoreType.DMA,
            pltpu.SemaphoreType.DMA,
        ],
        compiler_params=pltpu.CompilerParams(
            dimension_semantics=("arbitrary", "arbitrary"),
            vmem_limit_bytes=48 * 1024 * 1024,
        ),
    )(idx_hbm, table3)

    h_t_emb = ht_out.reshape(b, n, 2, es)
    qual_rel_emb = rel_out.reshape(b, n, n_pairs, es)
    qual_ent_emb = ent_out.reshape(b, n, n_pairs, es)
    return h_t_emb, qual_rel_emb, qual_ent_emb
